# Initial kernel scaffold; baseline (speedup 1.0000x reference)
import jax
import jax.numpy as jnp
from jax.experimental import pallas as pl


def kernel(x0, x1, edge_index0, edge_index1, inter_edge_index, W_self0, W_neigh0, W_self1, W_neigh1, W_self_i, W_neigh_i):
    raise NotImplementedError("write your pallas kernel here")


# speedup vs baseline: 7.5641x; 7.5641x over previous
import jax
import jax.numpy as jnp
from jax import lax
from jax.experimental import pallas as pl
from jax.experimental.pallas import tpu as pltpu
from jax.experimental.pallas import tpu_sc as plsc

N0, E0 = 10000, 320000
N1, E1 = 2500, 40000
D = 128
NC, NS = 2, 16
NW = NC * NS

N1P = 2560
NCNT = 2512
PAD1 = 2500
ND0 = 10016
PAD0 = 10000

EPT0 = E0 // NW
EPT1 = E1 // NW
B0 = 8
J0, C0 = 10, 125
J1, C1 = 10, 125
JI, CI = 4, 80

_SC_PARAMS = pltpu.CompilerParams(needs_layout_passes=False,
                                  use_tc_tiling_on_sc=False)


def _zero_vec16():
    return jnp.zeros((16,), jnp.float32)


def _zero_2d(ref, rows):
    def _zrow(r, _):
        for k in range(8):
            ref[r, pl.ds(k * 16, 16)] = _zero_vec16()
        return 0
    lax.fori_loop(0, rows, _zrow, 0)


def _zero_1d(ref, n):
    def _z(i, _):
        ref[pl.ds(i * 16, 16)] = _zero_vec16()
        return 0
    lax.fori_loop(0, n // 16, _z, 0)


def _sc_g0_body(x0h, se0h, de0h, d0lh,
                p0o, d0o,
                acc0, se0v, de0v, d0lv, buf, deg0):
    cid = lax.axis_index("c")
    sid = lax.axis_index("s")
    wid = cid * NS + sid

    _zero_2d(buf, C0)
    _zero_1d(deg0, ND0)
    for t in range(5):
        pltpu.sync_copy(buf, acc0.at[pl.ds(sid * 625 + t * 125, 125)])
    plsc.subcore_barrier()

    ones16 = jnp.ones((16,), jnp.float32)

    def _block(b, _):
        pltpu.sync_copy(se0h.at[wid, b], se0v)
        pltpu.sync_copy(de0h.at[wid, b], de0v)
        pltpu.sync_copy(d0lh.at[wid, b], d0lv)

        def _g(j, _):
            pltpu.sync_copy(x0h.at[se0v.at[j]], buf)
            pltpu.sync_copy(buf, acc0.at[de0v.at[j]], add=True)
            return 0
        lax.fori_loop(0, J0, _g, 0)

        def _h(t, _):
            plsc.addupdate_scatter(deg0, [d0lv[t]], ones16)
            return 0
        lax.fori_loop(0, 79, _h, 0)
        return 0
    lax.fori_loop(0, B0, _block, 0)

    plsc.subcore_barrier()
    pltpu.sync_copy(acc0.at[pl.ds(sid * 625, 625)],
                    p0o.at[cid, pl.ds(sid * 625, 625)])
    pltpu.sync_copy(deg0, d0o.at[wid])


def _sc_g1_body(x1h, se1h, de1h, d1lh, cllh,
                p1o, d1o, cnto,
                acc1, se1v, de1v, d1lv, cllv, buf, deg1, cnt):
    cid = lax.axis_index("c")
    sid = lax.axis_index("s")
    wid = cid * NS + sid

    _zero_2d(buf, C1)
    _zero_1d(deg1, NCNT)
    _zero_1d(cnt, NCNT)
    pltpu.sync_copy(buf, acc1.at[pl.ds(sid * 160, 125)])
    pltpu.sync_copy(buf.at[pl.ds(0, 35)], acc1.at[pl.ds(sid * 160 + 125, 35)])

    pltpu.sync_copy(se1h.at[wid], se1v)
    pltpu.sync_copy(de1h.at[wid], de1v)
    pltpu.sync_copy(d1lh.at[wid], d1lv)
    pltpu.sync_copy(cllh.at[wid], cllv)
    plsc.subcore_barrier()

    def _g(j, _):
        pltpu.sync_copy(x1h.at[se1v.at[j]], buf)
        pltpu.sync_copy(buf, acc1.at[de1v.at[j]], add=True)
        return 0
    lax.fori_loop(0, J1, _g, 0)

    ones16 = jnp.ones((16,), jnp.float32)

    def _h(t, _):
        plsc.addupdate_scatter(deg1, [d1lv[t]], ones16)
        return 0
    lax.fori_loop(0, 79, _h, 0)

    def _hc(t, _):
        plsc.addupdate_scatter(cnt, [cllv[t]], ones16)
        return 0
    lax.fori_loop(0, 20, _hc, 0)

    plsc.subcore_barrier()
    pltpu.sync_copy(acc1.at[pl.ds(sid * 160, 160)],
                    p1o.at[cid, pl.ds(sid * 160, 160)])
    pltpu.sync_copy(deg1, d1o.at[wid])
    pltpu.sync_copy(cnt, cnto.at[wid])


def _sc_inter_body(z0h, z1h, clch, go, c2o,
                   accc, clv, rbuf, gbuf):
    cid = lax.axis_index("c")
    sid = lax.axis_index("s")
    wid = cid * NS + sid

    _zero_2d(rbuf, CI)
    pltpu.sync_copy(rbuf, accc.at[pl.ds(sid * 160, 80)])
    pltpu.sync_copy(rbuf, accc.at[pl.ds(sid * 160 + 80, 80)])
    pltpu.sync_copy(clch.at[wid], clv)
    plsc.subcore_barrier()

    for j in range(JI):
        base = wid * (JI * CI) + j * CI

        @pl.when(base < N0)
        def _(base=base, j=j):
            pltpu.sync_copy(z0h.at[pl.ds(base, CI)], rbuf)
            pltpu.sync_copy(rbuf, accc.at[clv.at[j]], add=True)
            pltpu.sync_copy(z1h.at[clv.at[j]], gbuf)
            pltpu.sync_copy(gbuf, go.at[pl.ds(base, CI)])

    plsc.subcore_barrier()
    pltpu.sync_copy(accc.at[pl.ds(sid * 160, 160)],
                    c2o.at[cid, pl.ds(sid * 160, 160)])


def _mm(a, w):
    return lax.dot_general(a, w, (((1,), (0,)), ((), ())),
                           preferred_element_type=jnp.float32)


def _lin_body(x_ref, w_ref, o_ref):
    o_ref[...] = _mm(x_ref[...], w_ref[...])


def _tc2a_body(p0_ref, d0_ref, xs_ref, wn_ref, wsi_ref, wni_ref,
               h_ref, s_ref, z_ref):
    a = p0_ref[0] + p0_ref[1]
    deg = jnp.sum(d0_ref[...], axis=0)[:N0]
    m = a / jnp.maximum(deg, 1.0)[:, None]
    h = jnp.maximum(xs_ref[...] + _mm(m, wn_ref[...]), 0.0)
    h_ref[...] = h
    s_ref[...] = _mm(h, wsi_ref[...])
    z_ref[...] = _mm(h, wni_ref[...])


def _tc2b_body(p1_ref, d1_ref, xs_ref, wn_ref, wsi_ref, wni_ref,
               h_ref, s_ref, z_ref):
    a = (p1_ref[0] + p1_ref[1])[:N1]
    deg = jnp.sum(d1_ref[...], axis=0)[:N1]
    m = a / jnp.maximum(deg, 1.0)[:, None]
    h = jnp.maximum(xs_ref[...] + _mm(m, wn_ref[...]), 0.0)
    h_ref[...] = h
    s_ref[...] = _mm(h, wsi_ref[...])
    z_ref[...] = _mm(h, wni_ref[...])


def _tc3a_body(x_ref, h_ref, s_ref, g_ref, o_ref):
    nz = jnp.maximum(s_ref[...] + g_ref[...], 0.0)
    o_ref[...] = x_ref[...] + 0.5 * (h_ref[...] + nz)


def _tc3b_body(x_ref, h_ref, s_ref, c2_ref, cnt_ref, o_ref):
    cnt = jnp.sum(cnt_ref[...], axis=0)[:N1]
    cs = (c2_ref[0] + c2_ref[1])[:N1]
    nz = jnp.maximum(s_ref[...] + cs / jnp.maximum(cnt, 1.0)[:, None], 0.0)
    o_ref[...] = x_ref[...] + 0.5 * (h_ref[...] + nz)


def kernel(x0, x1, edge_index0, edge_index1, inter_edge_index,
           W_self0, W_neigh0, W_self1, W_neigh1, W_self_i, W_neigh_i):
    f32 = jnp.float32
    i32 = jnp.int32

    src0 = edge_index0[0].reshape(NW, B0, J0, C0)
    dst0 = edge_index0[1].reshape(NW, B0, J0, C0)
    dst0l = jnp.pad(edge_index0[1].reshape(NW, B0, J0 * C0),
                    ((0, 0), (0, 0), (0, 14)),
                    constant_values=PAD0).reshape(NW, B0, 79, 16)
    src1 = edge_index1[0].reshape(NW, J1, C1)
    dst1 = edge_index1[1].reshape(NW, J1, C1)
    dst1l = jnp.pad(edge_index1[1].reshape(NW, EPT1), ((0, 0), (0, 14)),
                    constant_values=PAD1).reshape(NW, 79, 16)
    cluster = inter_edge_index[1, :N0] - N0
    cll = jnp.pad(cluster, (0, NW * JI * CI - N0),
                  constant_values=PAD1).reshape(NW, 20, 16)
    clc = jnp.pad(cluster, (0, NW * JI * CI - N0)).reshape(NW, JI, CI)

    mesh = plsc.VectorSubcoreMesh(core_axis_name="c", subcore_axis_name="s")

    p0, d0p = pl.kernel(
        _sc_g0_body,
        out_type=[
            jax.ShapeDtypeStruct((NC, N0, D), f32),
            jax.ShapeDtypeStruct((NW, ND0), f32),
        ],
        mesh=mesh,
        scratch_types=[
            pltpu.VMEM_SHARED((N0, D), f32),
            pltpu.VMEM((J0, C0), i32),
            pltpu.VMEM((J0, C0), i32),
            pltpu.VMEM((79, 16), i32),
            pltpu.VMEM((C0, D), f32),
            pltpu.VMEM((ND0,), f32),
        ],
        compiler_params=_SC_PARAMS,
        name="sc_g0_agg",
    )(x0, src0, dst0, dst0l)

    p1, d1p, cntp = pl.kernel(
        _sc_g1_body,
        out_type=[
            jax.ShapeDtypeStruct((NC, N1P, D), f32),
            jax.ShapeDtypeStruct((NW, NCNT), f32),
            jax.ShapeDtypeStruct((NW, NCNT), f32),
        ],
        mesh=mesh,
        scratch_types=[
            pltpu.VMEM_SHARED((N1P, D), f32),
            pltpu.VMEM((J1, C1), i32),
            pltpu.VMEM((J1, C1), i32),
            pltpu.VMEM((79, 16), i32),
            pltpu.VMEM((20, 16), i32),
            pltpu.VMEM((C1, D), f32),
            pltpu.VMEM((NCNT,), f32),
            pltpu.VMEM((NCNT,), f32),
        ],
        compiler_params=_SC_PARAMS,
        name="sc_g1_agg",
    )(x1, src1, dst1, dst1l, cll)

    xs0 = pl.pallas_call(
        _lin_body,
        grid=(5,),
        in_specs=[pl.BlockSpec((2000, D), lambda i: (i, 0)),
                  pl.BlockSpec((D, D), lambda i: (0, 0))],
        out_specs=pl.BlockSpec((2000, D), lambda i: (i, 0)),
        out_shape=jax.ShapeDtypeStruct((N0, D), f32),
    )(x0, W_self0)
    xs1 = pl.pallas_call(
        _lin_body,
        out_shape=jax.ShapeDtypeStruct((N1, D), f32),
    )(x1, W_self1)

    h0, s0, z0 = pl.pallas_call(
        _tc2a_body,
        out_shape=[jax.ShapeDtypeStruct((N0, D), f32),
                   jax.ShapeDtypeStruct((N0, D), f32),
                   jax.ShapeDtypeStruct((N0, D), f32)],
        compiler_params=pltpu.CompilerParams(vmem_limit_bytes=100 * 1024 * 1024),
    )(p0, d0p, xs0, W_neigh0, W_self_i, W_neigh_i)

    h1, s1, z1 = pl.pallas_call(
        _tc2b_body,
        out_shape=[jax.ShapeDtypeStruct((N1, D), f32),
                   jax.ShapeDtypeStruct((N1, D), f32),
                   jax.ShapeDtypeStruct((N1, D), f32)],
    )(p1, d1p, xs1, W_neigh1, W_self_i, W_neigh_i)

    g, c2 = pl.kernel(
        _sc_inter_body,
        out_type=[
            jax.ShapeDtypeStruct((N0, D), f32),
            jax.ShapeDtypeStruct((NC, N1P, D), f32),
        ],
        mesh=mesh,
        scratch_types=[
            pltpu.VMEM_SHARED((N1P, D), f32),
            pltpu.VMEM((JI, CI), i32),
            pltpu.VMEM((CI, D), f32),
            pltpu.VMEM((CI, D), f32),
        ],
        compiler_params=_SC_PARAMS,
        name="sc_inter",
    )(z0, z1, clc)

    out0 = pl.pallas_call(
        _tc3a_body,
        grid=(5,),
        in_specs=[pl.BlockSpec((2000, D), lambda i: (i, 0))] * 4,
        out_specs=pl.BlockSpec((2000, D), lambda i: (i, 0)),
        out_shape=jax.ShapeDtypeStruct((N0, D), f32),
    )(x0, h0, s0, g)

    out1 = pl.pallas_call(
        _tc3b_body,
        out_shape=jax.ShapeDtypeStruct((N1, D), f32),
    )(x1, h1, s1, c2, cntp)

    return (out0, out1)


# speedup vs baseline: 9.1569x; 1.2106x over previous
import jax
import jax.numpy as jnp
from jax import lax
from jax.experimental import pallas as pl
from jax.experimental.pallas import tpu as pltpu
from jax.experimental.pallas import tpu_sc as plsc

N0, E0 = 10000, 320000
N1, E1 = 2500, 40000
D = 128
NC, NS = 2, 16
NW = NC * NS

N1P = 2560
NCNT = 2512
PAD1 = 2500
ND0 = 10016
PAD0 = 10000

EPT0 = E0 // NW
EPT1 = E1 // NW
B0 = 8
J0, C0 = 10, 125
J1, C1 = 10, 125
JI, CI = 4, 80

_SC_PARAMS = pltpu.CompilerParams(needs_layout_passes=False,
                                  use_tc_tiling_on_sc=False)


def _zero_vec16():
    return jnp.zeros((16,), jnp.float32)


def _zero_2d(ref, rows):
    def _zrow(r, _):
        for k in range(8):
            ref[r, pl.ds(k * 16, 16)] = _zero_vec16()
        return 0
    lax.fori_loop(0, rows, _zrow, 0)


def _zero_1d(ref, n):
    def _z(i, _):
        ref[pl.ds(i * 16, 16)] = _zero_vec16()
        return 0
    lax.fori_loop(0, n // 16, _z, 0)


def _sc_g0_body(x0h, se0h, de0h, d0lh,
                p0o, d0o,
                acc0, sev0, sev1, dev0, dev1, dlv0, dlv1,
                bufa, bufb, deg0,
                gsa, gsb, ssa, ssb, isem):
    cid = lax.axis_index("c")
    sid = lax.axis_index("s")
    wid = cid * NS + sid

    sev = [sev0, sev1]
    dev = [dev0, dev1]
    dlv = [dlv0, dlv1]
    bufs = [bufa, bufb]
    gsem = [gsa, gsb]
    ssem = [ssa, ssb]

    _zero_2d(bufa, C0)
    _zero_1d(deg0, ND0)
    for t in range(5):
        pltpu.sync_copy(bufa, acc0.at[pl.ds(sid * 625 + t * 125, 125)])
    plsc.subcore_barrier()

    ones16 = jnp.ones((16,), jnp.float32)

    pltpu.sync_copy(se0h.at[wid, 0], sev[0])
    pltpu.sync_copy(de0h.at[wid, 0], dev[0])
    pltpu.sync_copy(d0lh.at[wid, 0], dlv[0])

    gd = [None, None]
    sd = [None, None]
    idxd = []
    gd[0] = pltpu.async_copy(x0h.at[sev[0].at[0]], bufs[0], gsem[0])
    for b in range(B0):
        ib = b % 2
        if b + 1 < B0:
            if sd[1] is not None:
                sd[1].wait()
                sd[1] = None
            nb = 1 - ib
            idxd = [pltpu.async_copy(se0h.at[wid, b + 1], sev[nb], isem),
                    pltpu.async_copy(de0h.at[wid, b + 1], dev[nb], isem),
                    pltpu.async_copy(d0lh.at[wid, b + 1], dlv[nb], isem)]
        for r in range(J0):
            j = b * J0 + r
            bsl = j % 2
            gd[bsl].wait()
            gd[bsl] = None
            if sd[1 - bsl] is not None:
                sd[1 - bsl].wait()
                sd[1 - bsl] = None
            if j + 1 < B0 * J0:
                if r == J0 - 1:
                    for dsc in idxd:
                        dsc.wait()
                    idxd = []
                    nib = (b + 1) % 2
                    nr = 0
                else:
                    nib = ib
                    nr = r + 1
                gd[1 - bsl] = pltpu.async_copy(
                    x0h.at[sev[nib].at[nr]], bufs[1 - bsl], gsem[1 - bsl])
            sd[bsl] = pltpu.async_copy(
                bufs[bsl], acc0.at[dev[ib].at[r]], ssem[bsl], add=True)

        def _h(t, _, ib=ib):
            plsc.addupdate_scatter(deg0, [dlv[ib][t]], ones16)
            return 0
        lax.fori_loop(0, 79, _h, 0)

    for k in range(2):
        if sd[k] is not None:
            sd[k].wait()

    plsc.subcore_barrier()
    pltpu.sync_copy(acc0.at[pl.ds(sid * 625, 625)],
                    p0o.at[cid, pl.ds(sid * 625, 625)])
    pltpu.sync_copy(deg0, d0o.at[wid])


def _sc_g1_body(x1h, se1h, de1h, d1lh, cllh,
                p1o, d1o, cnto,
                acc1, se1v, de1v, d1lv, cllv, buf, deg1, cnt):
    cid = lax.axis_index("c")
    sid = lax.axis_index("s")
    wid = cid * NS + sid

    _zero_2d(buf, C1)
    _zero_1d(deg1, NCNT)
    _zero_1d(cnt, NCNT)
    pltpu.sync_copy(buf, acc1.at[pl.ds(sid * 160, 125)])
    pltpu.sync_copy(buf.at[pl.ds(0, 35)], acc1.at[pl.ds(sid * 160 + 125, 35)])

    pltpu.sync_copy(se1h.at[wid], se1v)
    pltpu.sync_copy(de1h.at[wid], de1v)
    pltpu.sync_copy(d1lh.at[wid], d1lv)
    pltpu.sync_copy(cllh.at[wid], cllv)
    plsc.subcore_barrier()

    def _g(j, _):
        pltpu.sync_copy(x1h.at[se1v.at[j]], buf)
        pltpu.sync_copy(buf, acc1.at[de1v.at[j]], add=True)
        return 0
    lax.fori_loop(0, J1, _g, 0)

    ones16 = jnp.ones((16,), jnp.float32)

    def _h(t, _):
        plsc.addupdate_scatter(deg1, [d1lv[t]], ones16)
        return 0
    lax.fori_loop(0, 79, _h, 0)

    def _hc(t, _):
        plsc.addupdate_scatter(cnt, [cllv[t]], ones16)
        return 0
    lax.fori_loop(0, 20, _hc, 0)

    plsc.subcore_barrier()
    pltpu.sync_copy(acc1.at[pl.ds(sid * 160, 160)],
                    p1o.at[cid, pl.ds(sid * 160, 160)])
    pltpu.sync_copy(deg1, d1o.at[wid])
    pltpu.sync_copy(cnt, cnto.at[wid])


def _sc_inter_body(z0h, z1h, clch, go, c2o,
                   accc, clv, rbuf, gbuf):
    cid = lax.axis_index("c")
    sid = lax.axis_index("s")
    wid = cid * NS + sid

    _zero_2d(rbuf, CI)
    pltpu.sync_copy(rbuf, accc.at[pl.ds(sid * 160, 80)])
    pltpu.sync_copy(rbuf, accc.at[pl.ds(sid * 160 + 80, 80)])
    pltpu.sync_copy(clch.at[wid], clv)
    plsc.subcore_barrier()

    for j in range(JI):
        base = wid * (JI * CI) + j * CI

        @pl.when(base < N0)
        def _(base=base, j=j):
            pltpu.sync_copy(z0h.at[pl.ds(base, CI)], rbuf)
            pltpu.sync_copy(rbuf, accc.at[clv.at[j]], add=True)
            pltpu.sync_copy(z1h.at[clv.at[j]], gbuf)
            pltpu.sync_copy(gbuf, go.at[pl.ds(base, CI)])

    plsc.subcore_barrier()
    pltpu.sync_copy(accc.at[pl.ds(sid * 160, 160)],
                    c2o.at[cid, pl.ds(sid * 160, 160)])


def _mm(a, w):
    return lax.dot_general(a, w, (((1,), (0,)), ((), ())),
                           preferred_element_type=jnp.float32)


def _lin_body(x_ref, w_ref, o_ref):
    o_ref[...] = _mm(x_ref[...], w_ref[...])


def _tc2a_body(p0_ref, d0_ref, xs_ref, wn_ref, wsi_ref, wni_ref,
               h_ref, s_ref, z_ref):
    a = p0_ref[0] + p0_ref[1]
    deg = jnp.sum(d0_ref[...], axis=0)[:N0]
    m = a / jnp.maximum(deg, 1.0)[:, None]
    h = jnp.maximum(xs_ref[...] + _mm(m, wn_ref[...]), 0.0)
    h_ref[...] = h
    s_ref[...] = _mm(h, wsi_ref[...])
    z_ref[...] = _mm(h, wni_ref[...])


def _tc2b_body(p1_ref, d1_ref, xs_ref, wn_ref, wsi_ref, wni_ref,
               h_ref, s_ref, z_ref):
    a = (p1_ref[0] + p1_ref[1])[:N1]
    deg = jnp.sum(d1_ref[...], axis=0)[:N1]
    m = a / jnp.maximum(deg, 1.0)[:, None]
    h = jnp.maximum(xs_ref[...] + _mm(m, wn_ref[...]), 0.0)
    h_ref[...] = h
    s_ref[...] = _mm(h, wsi_ref[...])
    z_ref[...] = _mm(h, wni_ref[...])


def _tc3a_body(x_ref, h_ref, s_ref, g_ref, o_ref):
    nz = jnp.maximum(s_ref[...] + g_ref[...], 0.0)
    o_ref[...] = x_ref[...] + 0.5 * (h_ref[...] + nz)


def _tc3b_body(x_ref, h_ref, s_ref, c2_ref, cnt_ref, o_ref):
    cnt = jnp.sum(cnt_ref[...], axis=0)[:N1]
    cs = (c2_ref[0] + c2_ref[1])[:N1]
    nz = jnp.maximum(s_ref[...] + cs / jnp.maximum(cnt, 1.0)[:, None], 0.0)
    o_ref[...] = x_ref[...] + 0.5 * (h_ref[...] + nz)


def kernel(x0, x1, edge_index0, edge_index1, inter_edge_index,
           W_self0, W_neigh0, W_self1, W_neigh1, W_self_i, W_neigh_i):
    f32 = jnp.float32
    i32 = jnp.int32

    src0 = edge_index0[0].reshape(NW, B0, J0, C0)
    dst0 = edge_index0[1].reshape(NW, B0, J0, C0)
    dst0l = jnp.pad(edge_index0[1].reshape(NW, B0, J0 * C0),
                    ((0, 0), (0, 0), (0, 14)),
                    constant_values=PAD0).reshape(NW, B0, 79, 16)
    src1 = edge_index1[0].reshape(NW, J1, C1)
    dst1 = edge_index1[1].reshape(NW, J1, C1)
    dst1l = jnp.pad(edge_index1[1].reshape(NW, EPT1), ((0, 0), (0, 14)),
                    constant_values=PAD1).reshape(NW, 79, 16)
    cluster = inter_edge_index[1, :N0] - N0
    cll = jnp.pad(cluster, (0, NW * JI * CI - N0),
                  constant_values=PAD1).reshape(NW, 20, 16)
    clc = jnp.pad(cluster, (0, NW * JI * CI - N0)).reshape(NW, JI, CI)

    mesh = plsc.VectorSubcoreMesh(core_axis_name="c", subcore_axis_name="s")

    p0, d0p = pl.kernel(
        _sc_g0_body,
        out_type=[
            jax.ShapeDtypeStruct((NC, N0, D), f32),
            jax.ShapeDtypeStruct((NW, ND0), f32),
        ],
        mesh=mesh,
        scratch_types=[
            pltpu.VMEM_SHARED((N0, D), f32),
            pltpu.VMEM((J0, C0), i32),
            pltpu.VMEM((J0, C0), i32),
            pltpu.VMEM((J0, C0), i32),
            pltpu.VMEM((J0, C0), i32),
            pltpu.VMEM((79, 16), i32),
            pltpu.VMEM((79, 16), i32),
            pltpu.VMEM((C0, D), f32),
            pltpu.VMEM((C0, D), f32),
            pltpu.VMEM((ND0,), f32),
            pltpu.SemaphoreType.DMA,
            pltpu.SemaphoreType.DMA,
            pltpu.SemaphoreType.DMA,
            pltpu.SemaphoreType.DMA,
            pltpu.SemaphoreType.DMA,
        ],
        compiler_params=_SC_PARAMS,
        name="sc_g0_agg",
    )(x0, src0, dst0, dst0l)

    p1, d1p, cntp = pl.kernel(
        _sc_g1_body,
        out_type=[
            jax.ShapeDtypeStruct((NC, N1P, D), f32),
            jax.ShapeDtypeStruct((NW, NCNT), f32),
            jax.ShapeDtypeStruct((NW, NCNT), f32),
        ],
        mesh=mesh,
        scratch_types=[
            pltpu.VMEM_SHARED((N1P, D), f32),
            pltpu.VMEM((J1, C1), i32),
            pltpu.VMEM((J1, C1), i32),
            pltpu.VMEM((79, 16), i32),
            pltpu.VMEM((20, 16), i32),
            pltpu.VMEM((C1, D), f32),
            pltpu.VMEM((NCNT,), f32),
            pltpu.VMEM((NCNT,), f32),
        ],
        compiler_params=_SC_PARAMS,
        name="sc_g1_agg",
    )(x1, src1, dst1, dst1l, cll)

    xs0 = pl.pallas_call(
        _lin_body,
        grid=(5,),
        in_specs=[pl.BlockSpec((2000, D), lambda i: (i, 0)),
                  pl.BlockSpec((D, D), lambda i: (0, 0))],
        out_specs=pl.BlockSpec((2000, D), lambda i: (i, 0)),
        out_shape=jax.ShapeDtypeStruct((N0, D), f32),
    )(x0, W_self0)
    xs1 = pl.pallas_call(
        _lin_body,
        out_shape=jax.ShapeDtypeStruct((N1, D), f32),
    )(x1, W_self1)

    h0, s0, z0 = pl.pallas_call(
        _tc2a_body,
        out_shape=[jax.ShapeDtypeStruct((N0, D), f32),
                   jax.ShapeDtypeStruct((N0, D), f32),
                   jax.ShapeDtypeStruct((N0, D), f32)],
        compiler_params=pltpu.CompilerParams(vmem_limit_bytes=100 * 1024 * 1024),
    )(p0, d0p, xs0, W_neigh0, W_self_i, W_neigh_i)

    h1, s1, z1 = pl.pallas_call(
        _tc2b_body,
        out_shape=[jax.ShapeDtypeStruct((N1, D), f32),
                   jax.ShapeDtypeStruct((N1, D), f32),
                   jax.ShapeDtypeStruct((N1, D), f32)],
    )(p1, d1p, xs1, W_neigh1, W_self_i, W_neigh_i)

    g, c2 = pl.kernel(
        _sc_inter_body,
        out_type=[
            jax.ShapeDtypeStruct((N0, D), f32),
            jax.ShapeDtypeStruct((NC, N1P, D), f32),
        ],
        mesh=mesh,
        scratch_types=[
            pltpu.VMEM_SHARED((N1P, D), f32),
            pltpu.VMEM((JI, CI), i32),
            pltpu.VMEM((CI, D), f32),
            pltpu.VMEM((CI, D), f32),
        ],
        compiler_params=_SC_PARAMS,
        name="sc_inter",
    )(z0, z1, clc)

    out0 = pl.pallas_call(
        _tc3a_body,
        grid=(5,),
        in_specs=[pl.BlockSpec((2000, D), lambda i: (i, 0))] * 4,
        out_specs=pl.BlockSpec((2000, D), lambda i: (i, 0)),
        out_shape=jax.ShapeDtypeStruct((N0, D), f32),
    )(x0, h0, s0, g)

    out1 = pl.pallas_call(
        _tc3b_body,
        out_shape=jax.ShapeDtypeStruct((N1, D), f32),
    )(x1, h1, s1, c2, cntp)

    return (out0, out1)
